# Initial kernel scaffold; baseline (speedup 1.0000x reference)
import jax
import jax.numpy as jnp
from jax.experimental import pallas as pl


def kernel(input, edge_index, W, a):
    raise NotImplementedError("write your pallas kernel here")


# speedup vs baseline: 22.6338x; 22.6338x over previous
import functools

import jax
import jax.numpy as jnp
from jax import lax
from jax.experimental import pallas as pl
from jax.experimental.pallas import tpu as pltpu
from jax.experimental.pallas import tpu_sc as plsc

N = 10000
F = 128
E = 320000
ALPHA = 0.2
NC = 2
NS = 16
NW = NC * NS
EPT = E // NW
B = 64
CPT = EPT // B
TAIL = EPT - CPT * B
NBUF = 2
ACC_T = 10
RPT = N // ACC_T


def _lrelu(t):
    return jnp.maximum(t, ALPHA * t)


def _k1_body(x_ref, w_ref, a1_ref, a2_ref, h_ref, s1_ref, s2_ref):
    hb = jnp.dot(x_ref[...], w_ref[...], preferred_element_type=jnp.float32)
    h_ref[...] = hb
    s1_ref[...] = jnp.dot(hb, a1_ref[...], preferred_element_type=jnp.float32)
    s2_ref[...] = jnp.dot(hb, a2_ref[...], preferred_element_type=jnp.float32)


_BN = 400


def _k1(x, W, a1, a2):
    grid = (N // _BN,)
    return pl.pallas_call(
        _k1_body,
        grid=grid,
        in_specs=[
            pl.BlockSpec((_BN, F), lambda i: (i, 0)),
            pl.BlockSpec((F, F), lambda i: (0, 0)),
            pl.BlockSpec((F, 1), lambda i: (0, 0)),
            pl.BlockSpec((F, 1), lambda i: (0, 0)),
        ],
        out_specs=[
            pl.BlockSpec((_BN, F), lambda i: (i, 0)),
            pl.BlockSpec((_BN, 1), lambda i: (i, 0)),
            pl.BlockSpec((_BN, 1), lambda i: (i, 0)),
        ],
        out_shape=[
            jax.ShapeDtypeStruct((N, F), jnp.float32),
            jax.ShapeDtypeStruct((N, 1), jnp.float32),
            jax.ShapeDtypeStruct((N, 1), jnp.float32),
        ],
    )(x, W, a1, a2)


def _k2_body(h_hbm, s1_hbm, s2_hbm, idx_hbm,
             den_out, acc_out,
             s1_v, s2_v, ibuf, uidx, rows_v, exA, exB,
             gsA, gsB, ssA, ssB,
             den_sh, acc_sh):
    cid = lax.axis_index("c")
    sid = lax.axis_index("s")
    w = sid * NC + cid
    exbufs = (exA, exB)
    gsems = (gsA, gsB)
    ssems = (ssA, ssB)

    @pl.when(sid == 0)
    def _():
        def zs(i, _):
            s1_v[pl.ds(i * 16, 16)] = jnp.zeros((16,), jnp.float32)
            return 0
        lax.fori_loop(0, N // 16, zs, 0)
        pltpu.sync_copy(s1_v, den_sh)

    pltpu.sync_copy(s1_hbm, s1_v)
    pltpu.sync_copy(s2_hbm, s2_v)
    pltpu.sync_copy(idx_hbm.at[pl.ds(pl.multiple_of(w * EPT, 8), EPT)], ibuf)

    def mred(i, m):
        return jnp.maximum(m, s2_v[pl.ds(i * 16, 16)])
    mvec = lax.fori_loop(0, N // 16, mred, jnp.full((16,), -3.4e38, jnp.float32))
    lanes = lax.iota(jnp.int32, 16)
    for step in (1, 2, 4, 8):
        exA[pl.ds(0, 16)] = mvec
        mvec = jnp.maximum(mvec, plsc.load_gather(exA, [lanes ^ step]))
    gmax = mvec

    def zero_rows(r, _):
        for j in range(F // 16):
            rows_v[0, r, pl.ds(j * 16, 16)] = jnp.zeros((16,), jnp.float32)
        return 0
    lax.fori_loop(0, B, zero_rows, 0)

    base = sid * RPT

    @pl.when(sid < ACC_T)
    def _():
        for off in range(0, RPT - B, B):
            pltpu.sync_copy(rows_v.at[0],
                            acc_sh.at[pl.ds(base + off, B)])
        done = ((RPT - B) // B + 1) * B
        pltpu.sync_copy(rows_v.at[0, pl.ds(0, RPT - done)],
                        acc_sh.at[pl.ds(base + done, RPT - done)])

    def unpack(c, ub, nun):
        def ug(g, _):
            p = ibuf[pl.ds(c * B + g * 16, 16)]
            uidx[ub, 0, pl.ds(g * 16, 16)] = p >> 16
            uidx[ub, 1, pl.ds(g * 16, 16)] = p & 0xFFFF
            return 0
        lax.fori_loop(0, nun, ug, 0)

    unpack(0, 0, B // 16)
    pltpu.async_copy(h_hbm.at[uidx.at[0, 1]], rows_v.at[0], gsems[0])

    plsc.subcore_barrier()

    def compute_chunk(b, nex=B // 16):
        exbuf = exbufs[b]

        def exgrp(g, _):
            isrc = uidx[b, 0, pl.ds(g * 16, 16)]
            idst = uidx[b, 1, pl.ds(g * 16, 16)]
            s1g = plsc.load_gather(s1_v, [isrc])
            s2g = plsc.load_gather(s2_v, [idst])
            e = _lrelu(s1g + s2g)
            sh = _lrelu(s1g + gmax)
            exbuf[pl.ds(g * 16, 16)] = jnp.exp(e - sh)
            return 0
        lax.fori_loop(0, nex, exgrp, 0)
        for g in range(nex, B // 16):
            exbuf[pl.ds(g * 16, 16)] = jnp.zeros((16,), jnp.float32)

        def scale(ei, _):
            exs = plsc.load_gather(exbuf, [jnp.zeros((16,), jnp.int32) + ei])
            for j in range(F // 16):
                rows_v[b, ei, pl.ds(j * 16, 16)] = (
                    rows_v[b, ei, pl.ds(j * 16, 16)] * exs)
            return 0
        lax.fori_loop(0, B, scale, 0)

    def issue_scatter(b):
        pltpu.async_copy(rows_v.at[b], acc_sh.at[uidx.at[b, 0]],
                         ssems[b], add=True)
        pltpu.async_copy(exbufs[b], den_sh.at[uidx.at[b, 0]],
                         ssems[b], add=True)

    def wait_scatter(b):
        pltpu.make_async_copy(rows_v.at[b], acc_sh.at[uidx.at[b, 0]],
                              ssems[b]).wait()
        pltpu.make_async_copy(exbufs[b], den_sh.at[uidx.at[b, 0]],
                              ssems[b]).wait()

    def outer(i, _):
        for b in range(NBUF):
            c = i * NBUF + b
            nb = (b + 1) % NBUF

            @pl.when(c + 1 < CPT)
            def _():
                @pl.when(c >= NBUF - 1)
                def _():
                    wait_scatter(nb)
                unpack(c + 1, nb, B // 16)
                pltpu.async_copy(h_hbm.at[uidx.at[nb, 1]],
                                 rows_v.at[nb], gsems[nb])

            pltpu.make_async_copy(h_hbm.at[uidx.at[b, 1]],
                                  rows_v.at[b], gsems[b]).wait()
            compute_chunk(b)
            issue_scatter(b)
        return 0
    lax.fori_loop(0, CPT // NBUF, outer, 0)

    for b in range(NBUF):
        wait_scatter(b)

    unpack(CPT, 0, TAIL // 16)
    zi = jnp.zeros((16,), jnp.int32)
    for g in range(TAIL // 16, B // 16):
        uidx[0, 0, pl.ds(g * 16, 16)] = zi
        uidx[0, 1, pl.ds(g * 16, 16)] = zi
    pltpu.async_copy(h_hbm.at[uidx.at[0, 1]], rows_v.at[0], gsems[0]).wait()
    compute_chunk(0, TAIL // 16)
    pltpu.sync_copy(rows_v.at[0], acc_sh.at[uidx.at[0, 0]], add=True)
    pltpu.sync_copy(exbufs[0], den_sh.at[uidx.at[0, 0]], add=True)

    plsc.subcore_barrier()

    @pl.when(sid < ACC_T)
    def _():
        pltpu.sync_copy(acc_sh.at[pl.ds(base, RPT)],
                        acc_out.at[cid, pl.ds(base, RPT)])

    @pl.when(sid == 0)
    def _():
        pltpu.sync_copy(den_sh, den_out.at[cid])


_k2 = functools.partial(
    pl.kernel,
    mesh=plsc.VectorSubcoreMesh(core_axis_name="c", subcore_axis_name="s"),
    out_type=[
        jax.ShapeDtypeStruct((NC, N), jnp.float32),
        jax.ShapeDtypeStruct((NC, N, F), jnp.float32),
    ],
    scratch_types=[
        pltpu.VMEM((N,), jnp.float32),
        pltpu.VMEM((N,), jnp.float32),
        pltpu.VMEM((EPT,), jnp.int32),
        pltpu.VMEM((NBUF, 2, B), jnp.int32),
        pltpu.VMEM((NBUF, B, F), jnp.float32),
        pltpu.VMEM((B,), jnp.float32),
        pltpu.VMEM((B,), jnp.float32),
        pltpu.SemaphoreType.DMA,
        pltpu.SemaphoreType.DMA,
        pltpu.SemaphoreType.DMA,
        pltpu.SemaphoreType.DMA,
        pltpu.VMEM_SHARED((N,), jnp.float32),
        pltpu.VMEM_SHARED((N, F), jnp.float32),
    ],
    compiler_params=pltpu.CompilerParams(needs_layout_passes=False),
)(_k2_body)


def _k3_body(acc_ref, den_ref, out_ref):
    s = acc_ref[0] + acc_ref[1]
    d = den_ref[0] + den_ref[1]
    hp = s / (d + 1e-16)
    out_ref[...] = jnp.where(hp > 0.0, hp,
                             jnp.exp(jnp.minimum(hp, 0.0)) - 1.0)


def _k3(acc, den):
    return pl.pallas_call(
        _k3_body,
        out_shape=jax.ShapeDtypeStruct((N, F), jnp.float32),
    )(acc, den)


@jax.jit
def kernel(input, edge_index, W, a):
    a1 = a[:F]
    a2 = a[F:]
    h, s1, s2 = _k1(input, W, a1, a2)
    packed = edge_index[0] * 65536 + edge_index[1]
    den, acc = _k2(h, s1.reshape(N), s2.reshape(N), packed)
    return _k3(acc, den[:, :, None])


# speedup vs baseline: 25.5323x; 1.1281x over previous
import functools

import jax
import jax.numpy as jnp
from jax import lax
from jax.experimental import pallas as pl
from jax.experimental.pallas import tpu as pltpu
from jax.experimental.pallas import tpu_sc as plsc

N = 10000
F = 128
E = 320000
ALPHA = 0.2
NC = 2
NS = 16
NW = NC * NS
EPT = E // NW
B = 64
CPT = EPT // B
TAIL = EPT - CPT * B
NBUF = 3
NIDX = 4
UNROLL = 12
ACC_T = 10
RPT = N // ACC_T


def _lrelu(t):
    return jnp.maximum(t, ALPHA * t)


def _k1_body(x_ref, w_ref, a1_ref, a2_ref, h_ref, s1_ref, s2_ref):
    hb = jnp.dot(x_ref[...], w_ref[...], preferred_element_type=jnp.float32)
    h_ref[...] = hb
    s1_ref[...] = jnp.dot(hb, a1_ref[...], preferred_element_type=jnp.float32)
    s2_ref[...] = jnp.dot(hb, a2_ref[...], preferred_element_type=jnp.float32)


_BN = 400


def _k1(x, W, a1, a2):
    grid = (N // _BN,)
    return pl.pallas_call(
        _k1_body,
        grid=grid,
        in_specs=[
            pl.BlockSpec((_BN, F), lambda i: (i, 0)),
            pl.BlockSpec((F, F), lambda i: (0, 0)),
            pl.BlockSpec((F, 1), lambda i: (0, 0)),
            pl.BlockSpec((F, 1), lambda i: (0, 0)),
        ],
        out_specs=[
            pl.BlockSpec((_BN, F), lambda i: (i, 0)),
            pl.BlockSpec((_BN, 1), lambda i: (i, 0)),
            pl.BlockSpec((_BN, 1), lambda i: (i, 0)),
        ],
        out_shape=[
            jax.ShapeDtypeStruct((N, F), jnp.float32),
            jax.ShapeDtypeStruct((N, 1), jnp.float32),
            jax.ShapeDtypeStruct((N, 1), jnp.float32),
        ],
    )(x, W, a1, a2)


def _k2_body(h_hbm, s1_hbm, s2_hbm, idx_hbm,
             den_out, acc_out,
             s1_v, s2_v, pidx, uidx, rows_v, exA, exB, exC,
             gsA, gsB, gsC, ssA, ssB, ssC, isA, isB, isC, isD,
             den_sh, acc_sh):
    cid = lax.axis_index("c")
    sid = lax.axis_index("s")
    w = sid * NC + cid
    ebase = w * EPT
    exbufs = (exA, exB, exC)
    gsems = (gsA, gsB, gsC)
    ssems = (ssA, ssB, ssC)
    isems = (isA, isB, isC, isD)

    def idx_src(c):
        return idx_hbm.at[pl.ds(pl.multiple_of(ebase + c * B, 8), B)]

    @pl.when(sid == 0)
    def _():
        def zs(i, _):
            s1_v[pl.ds(i * 16, 16)] = jnp.zeros((16,), jnp.float32)
            return 0
        lax.fori_loop(0, N // 16, zs, 0)
        pltpu.sync_copy(s1_v, den_sh)

    pltpu.sync_copy(s1_hbm, s1_v)
    pltpu.sync_copy(s2_hbm, s2_v)

    def mred(i, m):
        return jnp.maximum(m, s2_v[pl.ds(i * 16, 16)])
    mvec = lax.fori_loop(0, N // 16, mred, jnp.full((16,), -3.4e38, jnp.float32))
    lanes = lax.iota(jnp.int32, 16)
    for step in (1, 2, 4, 8):
        exA[pl.ds(0, 16)] = mvec
        mvec = jnp.maximum(mvec, plsc.load_gather(exA, [lanes ^ step]))
    gmax = mvec

    def zero_rows(r, _):
        for j in range(F // 16):
            rows_v[0, r, pl.ds(j * 16, 16)] = jnp.zeros((16,), jnp.float32)
        return 0
    lax.fori_loop(0, B, zero_rows, 0)

    base = sid * RPT

    @pl.when(sid < ACC_T)
    def _():
        for off in range(0, RPT - B, B):
            pltpu.sync_copy(rows_v.at[0],
                            acc_sh.at[pl.ds(base + off, B)])
        done = ((RPT - B) // B + 1) * B
        pltpu.sync_copy(rows_v.at[0, pl.ds(0, RPT - done)],
                        acc_sh.at[pl.ds(base + done, RPT - done)])

    def unpack(ps, ub, nun):
        for g in range(nun):
            p = pidx[ps, pl.ds(g * 16, 16)]
            uidx[ub, 0, pl.ds(g * 16, 16)] = p >> 16
            uidx[ub, 1, pl.ds(g * 16, 16)] = p & 0xFFFF

    for c0 in range(NIDX - 1):
        pltpu.async_copy(idx_src(c0), pidx.at[c0], isems[c0])
    pltpu.make_async_copy(idx_src(0), pidx.at[0], isems[0]).wait()
    unpack(0, 0, B // 16)
    pltpu.async_copy(h_hbm.at[uidx.at[0, 1]], rows_v.at[0], gsems[0])

    plsc.subcore_barrier()

    def compute_chunk(b, nex=B // 16):
        exbuf = exbufs[b]

        def exgrp(g, _):
            isrc = uidx[b, 0, pl.ds(g * 16, 16)]
            idst = uidx[b, 1, pl.ds(g * 16, 16)]
            s1g = plsc.load_gather(s1_v, [isrc])
            s2g = plsc.load_gather(s2_v, [idst])
            e = _lrelu(s1g + s2g)
            sh = _lrelu(s1g + gmax)
            exbuf[pl.ds(g * 16, 16)] = jnp.exp(e - sh)
            return 0
        lax.fori_loop(0, nex, exgrp, 0)
        for g in range(nex, B // 16):
            exbuf[pl.ds(g * 16, 16)] = jnp.zeros((16,), jnp.float32)

        def scale(ei, _):
            exs = plsc.load_gather(exbuf, [jnp.zeros((16,), jnp.int32) + ei])
            for j in range(F // 16):
                rows_v[b, ei, pl.ds(j * 16, 16)] = (
                    rows_v[b, ei, pl.ds(j * 16, 16)] * exs)
            return 0
        lax.fori_loop(0, B, scale, 0)

    def issue_scatter(b):
        pltpu.async_copy(rows_v.at[b], acc_sh.at[uidx.at[b, 0]],
                         ssems[b], add=True)
        pltpu.async_copy(exbufs[b], den_sh.at[uidx.at[b, 0]],
                         ssems[b], add=True)

    def wait_scatter(b):
        pltpu.make_async_copy(rows_v.at[b], acc_sh.at[uidx.at[b, 0]],
                              ssems[b]).wait()
        pltpu.make_async_copy(exbufs[b], den_sh.at[uidx.at[b, 0]],
                              ssems[b]).wait()

    def outer(i, _):
        for k in range(UNROLL):
            c = i * UNROLL + k
            s3 = k % NBUF
            n3 = (k + 1) % NBUF
            n4 = (k + 1) % NIDX
            p4 = (k + NIDX - 1) % NIDX

            @pl.when(c + NIDX - 1 < CPT)
            def _():
                pltpu.async_copy(idx_src(c + NIDX - 1), pidx.at[p4],
                                 isems[p4])

            @pl.when(c + 1 < CPT)
            def _():
                @pl.when(c >= NBUF - 1)
                def _():
                    wait_scatter(n3)
                pltpu.make_async_copy(idx_src(c + 1), pidx.at[n4],
                                      isems[n4]).wait()
                unpack(n4, n3, B // 16)
                pltpu.async_copy(h_hbm.at[uidx.at[n3, 1]],
                                 rows_v.at[n3], gsems[n3])

            pltpu.make_async_copy(h_hbm.at[uidx.at[s3, 1]],
                                  rows_v.at[s3], gsems[s3]).wait()
            compute_chunk(s3)
            issue_scatter(s3)
        return 0
    lax.fori_loop(0, CPT // UNROLL, outer, 0)

    for b in range(NBUF):
        wait_scatter(b)

    pltpu.sync_copy(idx_hbm.at[pl.ds(pl.multiple_of(ebase + CPT * B, 8), TAIL)],
                    pidx.at[0, pl.ds(0, TAIL)])
    unpack(0, 0, TAIL // 16)
    zi = jnp.zeros((16,), jnp.int32)
    for g in range(TAIL // 16, B // 16):
        uidx[0, 0, pl.ds(g * 16, 16)] = zi
        uidx[0, 1, pl.ds(g * 16, 16)] = zi
    pltpu.async_copy(h_hbm.at[uidx.at[0, 1]], rows_v.at[0], gsems[0]).wait()
    compute_chunk(0, TAIL // 16)
    pltpu.sync_copy(rows_v.at[0], acc_sh.at[uidx.at[0, 0]], add=True)
    pltpu.sync_copy(exbufs[0], den_sh.at[uidx.at[0, 0]], add=True)

    plsc.subcore_barrier()

    @pl.when(sid < ACC_T)
    def _():
        pltpu.sync_copy(acc_sh.at[pl.ds(base, RPT)],
                        acc_out.at[cid, pl.ds(base, RPT)])

    @pl.when(sid == 0)
    def _():
        pltpu.sync_copy(den_sh, den_out.at[cid])


_k2 = functools.partial(
    pl.kernel,
    mesh=plsc.VectorSubcoreMesh(core_axis_name="c", subcore_axis_name="s"),
    out_type=[
        jax.ShapeDtypeStruct((NC, N), jnp.float32),
        jax.ShapeDtypeStruct((NC, N, F), jnp.float32),
    ],
    scratch_types=[
        pltpu.VMEM((N,), jnp.float32),
        pltpu.VMEM((N,), jnp.float32),
        pltpu.VMEM((NIDX, B), jnp.int32),
        pltpu.VMEM((NBUF, 2, B), jnp.int32),
        pltpu.VMEM((NBUF, B, F), jnp.float32),
        pltpu.VMEM((B,), jnp.float32),
        pltpu.VMEM((B,), jnp.float32),
        pltpu.VMEM((B,), jnp.float32),
        pltpu.SemaphoreType.DMA,
        pltpu.SemaphoreType.DMA,
        pltpu.SemaphoreType.DMA,
        pltpu.SemaphoreType.DMA,
        pltpu.SemaphoreType.DMA,
        pltpu.SemaphoreType.DMA,
        pltpu.SemaphoreType.DMA,
        pltpu.SemaphoreType.DMA,
        pltpu.SemaphoreType.DMA,
        pltpu.SemaphoreType.DMA,
        pltpu.VMEM_SHARED((N,), jnp.float32),
        pltpu.VMEM_SHARED((N, F), jnp.float32),
    ],
    compiler_params=pltpu.CompilerParams(needs_layout_passes=False),
)(_k2_body)


def _k3_body(acc_ref, den_ref, out_ref):
    s = acc_ref[0] + acc_ref[1]
    d = den_ref[0] + den_ref[1]
    hp = s / (d + 1e-16)
    out_ref[...] = jnp.where(hp > 0.0, hp,
                             jnp.exp(jnp.minimum(hp, 0.0)) - 1.0)


def _k3(acc, den):
    return pl.pallas_call(
        _k3_body,
        out_shape=jax.ShapeDtypeStruct((N, F), jnp.float32),
    )(acc, den)


@jax.jit
def kernel(input, edge_index, W, a):
    a1 = a[:F]
    a2 = a[F:]
    h, s1, s2 = _k1(input, W, a1, a2)
    packed = edge_index[0] * 65536 + edge_index[1]
    den, acc = _k2(h, s1.reshape(N), s2.reshape(N), packed)
    return _k3(acc, den[:, :, None])


# speedup vs baseline: 25.6687x; 1.0053x over previous
import functools

import jax
import jax.numpy as jnp
from jax import lax
from jax.experimental import pallas as pl
from jax.experimental.pallas import tpu as pltpu
from jax.experimental.pallas import tpu_sc as plsc

N = 10000
F = 128
E = 320000
ALPHA = 0.2
NC = 2
NS = 16
NW = NC * NS
EPT = E // NW
B = 64
CPT = EPT // B
TAIL = EPT - CPT * B
NBUF = 3
NIDX = 4
UNROLL = 12
ACC_T = 10
RPT = N // ACC_T


def _lrelu(t):
    return jnp.maximum(t, ALPHA * t)


def _k1_body(x_ref, w_ref, a1_ref, a2_ref, h_ref, s1_ref, s2_ref):
    hb = jnp.dot(x_ref[...], w_ref[...], preferred_element_type=jnp.float32)
    h_ref[...] = hb
    s1_ref[...] = jnp.dot(hb, a1_ref[...], preferred_element_type=jnp.float32)
    s2_ref[...] = jnp.dot(hb, a2_ref[...], preferred_element_type=jnp.float32)


_BN = 400


def _k1(x, W, a1, a2):
    grid = (N // _BN,)
    return pl.pallas_call(
        _k1_body,
        grid=grid,
        in_specs=[
            pl.BlockSpec((_BN, F), lambda i: (i, 0)),
            pl.BlockSpec((F, F), lambda i: (0, 0)),
            pl.BlockSpec((F, 1), lambda i: (0, 0)),
            pl.BlockSpec((F, 1), lambda i: (0, 0)),
        ],
        out_specs=[
            pl.BlockSpec((_BN, F), lambda i: (i, 0)),
            pl.BlockSpec((_BN, 1), lambda i: (i, 0)),
            pl.BlockSpec((_BN, 1), lambda i: (i, 0)),
        ],
        out_shape=[
            jax.ShapeDtypeStruct((N, F), jnp.float32),
            jax.ShapeDtypeStruct((N, 1), jnp.float32),
            jax.ShapeDtypeStruct((N, 1), jnp.float32),
        ],
    )(x, W, a1, a2)


def _k2_body(h_hbm, s1_hbm, s2_hbm, idx_hbm,
             den_out, acc_out,
             s1_v, s2_v, pidx, uidx, rows_v, exA, exB, exC,
             gsA, gsB, gsC, ssA, ssB, ssC, isA, isB, isC, isD,
             den_sh, acc_sh):
    cid = lax.axis_index("c")
    sid = lax.axis_index("s")
    w = sid * NC + cid
    ebase = w * EPT
    exbufs = (exA, exB, exC)
    gsems = (gsA, gsB, gsC)
    ssems = (ssA, ssB, ssC)
    isems = (isA, isB, isC, isD)

    def idx_src(c):
        return idx_hbm.at[pl.ds(pl.multiple_of(ebase + c * B, 8), B)]

    @pl.when(sid == 0)
    def _():
        def zs(i, _):
            s1_v[pl.ds(i * 16, 16)] = jnp.zeros((16,), jnp.float32)
            return 0
        lax.fori_loop(0, N // 16, zs, 0)
        pltpu.sync_copy(s1_v, den_sh)

    pltpu.sync_copy(s1_hbm, s1_v)
    pltpu.sync_copy(s2_hbm, s2_v)

    def mred(i, m):
        return jnp.maximum(m, s2_v[pl.ds(i * 16, 16)])
    mvec = lax.fori_loop(0, N // 16, mred, jnp.full((16,), -3.4e38, jnp.float32))
    lanes = lax.iota(jnp.int32, 16)
    for step in (1, 2, 4, 8):
        exA[pl.ds(0, 16)] = mvec
        mvec = jnp.maximum(mvec, plsc.load_gather(exA, [lanes ^ step]))
    gmax = mvec

    def zero_rows(r, _):
        for j in range(F // 16):
            rows_v[0, r, pl.ds(j * 16, 16)] = jnp.zeros((16,), jnp.float32)
        return 0
    lax.fori_loop(0, B, zero_rows, 0)

    base = sid * RPT

    @pl.when(sid < ACC_T)
    def _():
        for off in range(0, RPT - B, B):
            pltpu.sync_copy(rows_v.at[0],
                            acc_sh.at[pl.ds(base + off, B)])
        done = ((RPT - B) // B + 1) * B
        pltpu.sync_copy(rows_v.at[0, pl.ds(0, RPT - done)],
                        acc_sh.at[pl.ds(base + done, RPT - done)])

    def unpack(ps, ub, nun):
        for g in range(nun):
            p = pidx[ps, pl.ds(g * 16, 16)]
            uidx[ub, 0, pl.ds(g * 16, 16)] = p >> 16
            uidx[ub, 1, pl.ds(g * 16, 16)] = p & 0xFFFF

    for c0 in range(NIDX - 1):
        pltpu.async_copy(idx_src(c0), pidx.at[c0], isems[c0])
    pltpu.make_async_copy(idx_src(0), pidx.at[0], isems[0]).wait()
    unpack(0, 0, B // 16)
    pltpu.async_copy(h_hbm.at[uidx.at[0, 1]], rows_v.at[0], gsems[0])

    plsc.subcore_barrier()

    def compute_chunk(b, nex=B // 16):
        exbuf = exbufs[b]

        def exgrp(g, _):
            isrc = uidx[b, 0, pl.ds(g * 16, 16)]
            idst = uidx[b, 1, pl.ds(g * 16, 16)]
            s1g = plsc.load_gather(s1_v, [isrc])
            s2g = plsc.load_gather(s2_v, [idst])
            e = _lrelu(s1g + s2g)
            sh = _lrelu(s1g + gmax)
            exbuf[pl.ds(g * 16, 16)] = jnp.exp(e - sh)
            return 0
        lax.fori_loop(0, nex, exgrp, 0)
        for g in range(nex, B // 16):
            exbuf[pl.ds(g * 16, 16)] = jnp.zeros((16,), jnp.float32)

        def scale(e2, _):
            for u in range(2):
                ei = e2 * 2 + u
                exs = plsc.load_gather(exbuf,
                                       [jnp.zeros((16,), jnp.int32) + ei])
                for j in range(F // 16):
                    rows_v[b, ei, pl.ds(j * 16, 16)] = (
                        rows_v[b, ei, pl.ds(j * 16, 16)] * exs)
            return 0
        lax.fori_loop(0, B // 2, scale, 0)

    def issue_scatter(b):
        pltpu.async_copy(rows_v.at[b], acc_sh.at[uidx.at[b, 0]],
                         ssems[b], add=True)
        pltpu.async_copy(exbufs[b], den_sh.at[uidx.at[b, 0]],
                         ssems[b], add=True)

    def wait_scatter(b):
        pltpu.make_async_copy(rows_v.at[b], acc_sh.at[uidx.at[b, 0]],
                              ssems[b]).wait()
        pltpu.make_async_copy(exbufs[b], den_sh.at[uidx.at[b, 0]],
                              ssems[b]).wait()

    def outer(i, _):
        for k in range(UNROLL):
            c = i * UNROLL + k
            s3 = k % NBUF
            n3 = (k + 1) % NBUF
            n4 = (k + 1) % NIDX
            p4 = (k + NIDX - 1) % NIDX

            @pl.when(c + NIDX - 1 < CPT)
            def _():
                pltpu.async_copy(idx_src(c + NIDX - 1), pidx.at[p4],
                                 isems[p4])

            @pl.when(c + 1 < CPT)
            def _():
                @pl.when(c >= NBUF - 1)
                def _():
                    wait_scatter(n3)
                pltpu.make_async_copy(idx_src(c + 1), pidx.at[n4],
                                      isems[n4]).wait()
                unpack(n4, n3, B // 16)
                pltpu.async_copy(h_hbm.at[uidx.at[n3, 1]],
                                 rows_v.at[n3], gsems[n3])

            pltpu.make_async_copy(h_hbm.at[uidx.at[s3, 1]],
                                  rows_v.at[s3], gsems[s3]).wait()
            compute_chunk(s3)
            issue_scatter(s3)
        return 0
    lax.fori_loop(0, CPT // UNROLL, outer, 0)

    for b in range(NBUF):
        wait_scatter(b)

    pltpu.sync_copy(idx_hbm.at[pl.ds(pl.multiple_of(ebase + CPT * B, 8), TAIL)],
                    pidx.at[0, pl.ds(0, TAIL)])
    unpack(0, 0, TAIL // 16)
    zi = jnp.zeros((16,), jnp.int32)
    for g in range(TAIL // 16, B // 16):
        uidx[0, 0, pl.ds(g * 16, 16)] = zi
        uidx[0, 1, pl.ds(g * 16, 16)] = zi
    pltpu.async_copy(h_hbm.at[uidx.at[0, 1]], rows_v.at[0], gsems[0]).wait()
    compute_chunk(0, TAIL // 16)
    pltpu.sync_copy(rows_v.at[0], acc_sh.at[uidx.at[0, 0]], add=True)
    pltpu.sync_copy(exbufs[0], den_sh.at[uidx.at[0, 0]], add=True)

    plsc.subcore_barrier()

    @pl.when(sid < ACC_T)
    def _():
        pltpu.sync_copy(acc_sh.at[pl.ds(base, RPT)],
                        acc_out.at[cid, pl.ds(base, RPT)])

    @pl.when(sid == 0)
    def _():
        pltpu.sync_copy(den_sh, den_out.at[cid])


_k2 = functools.partial(
    pl.kernel,
    mesh=plsc.VectorSubcoreMesh(core_axis_name="c", subcore_axis_name="s"),
    out_type=[
        jax.ShapeDtypeStruct((NC, N), jnp.float32),
        jax.ShapeDtypeStruct((NC, N, F), jnp.float32),
    ],
    scratch_types=[
        pltpu.VMEM((N,), jnp.float32),
        pltpu.VMEM((N,), jnp.float32),
        pltpu.VMEM((NIDX, B), jnp.int32),
        pltpu.VMEM((NBUF, 2, B), jnp.int32),
        pltpu.VMEM((NBUF, B, F), jnp.float32),
        pltpu.VMEM((B,), jnp.float32),
        pltpu.VMEM((B,), jnp.float32),
        pltpu.VMEM((B,), jnp.float32),
        pltpu.SemaphoreType.DMA,
        pltpu.SemaphoreType.DMA,
        pltpu.SemaphoreType.DMA,
        pltpu.SemaphoreType.DMA,
        pltpu.SemaphoreType.DMA,
        pltpu.SemaphoreType.DMA,
        pltpu.SemaphoreType.DMA,
        pltpu.SemaphoreType.DMA,
        pltpu.SemaphoreType.DMA,
        pltpu.SemaphoreType.DMA,
        pltpu.VMEM_SHARED((N,), jnp.float32),
        pltpu.VMEM_SHARED((N, F), jnp.float32),
    ],
    compiler_params=pltpu.CompilerParams(needs_layout_passes=False),
)(_k2_body)


def _k3_body(acc_ref, den_ref, out_ref):
    s = acc_ref[0] + acc_ref[1]
    d = den_ref[0] + den_ref[1]
    hp = s / (d + 1e-16)
    out_ref[...] = jnp.where(hp > 0.0, hp,
                             jnp.exp(jnp.minimum(hp, 0.0)) - 1.0)


def _k3(acc, den):
    return pl.pallas_call(
        _k3_body,
        out_shape=jax.ShapeDtypeStruct((N, F), jnp.float32),
    )(acc, den)


@jax.jit
def kernel(input, edge_index, W, a):
    a1 = a[:F]
    a2 = a[F:]
    h, s1, s2 = _k1(input, W, a1, a2)
    packed = edge_index[0] * 65536 + edge_index[1]
    den, acc = _k2(h, s1.reshape(N), s2.reshape(N), packed)
    return _k3(acc, den[:, :, None])
